# baseline (speedup 1.0000x reference)
import functools

import jax
import jax.numpy as jnp
from jax import lax
from jax.experimental import pallas as pl
from jax.experimental.pallas import tpu as pltpu
from jax.experimental.pallas import tpu_sc as plsc

_NUM_CORES = 2
_NUM_SUBCORES = 16
_NW = _NUM_CORES * _NUM_SUBCORES
_GCHUNK = 128

_VT = 1024
_NCHUNK = 8


def _gather_sum_sc(idx_flat, emb, B, C, D):
  per_w = B // _NW
  n_idx = per_w * C
  n_full = n_idx // _GCHUNK
  tail = n_idx - n_full * _GCHUNK

  mesh = plsc.VectorSubcoreMesh(
      core_axis_name="c", subcore_axis_name="s",
      num_cores=_NUM_CORES, num_subcores=_NUM_SUBCORES)

  @functools.partial(
      pl.kernel,
      out_type=jax.ShapeDtypeStruct((B, D), jnp.float32),
      mesh=mesh,
      compiler_params=pltpu.CompilerParams(use_tc_tiling_on_sc=False),
      scratch_types=[
          pltpu.VMEM((n_idx,), jnp.int32),
          pltpu.VMEM((n_idx, D), jnp.float32),
          pltpu.VMEM((per_w, D), jnp.float32),
          pltpu.SemaphoreType.DMA,
      ],
  )
  def gather_sum(emb_hbm, idx_hbm, out_hbm, idx_v, rows_v, acc_v, sem):
    wid = lax.axis_index("s") * _NUM_CORES + lax.axis_index("c")
    base = wid * n_idx
    pltpu.sync_copy(idx_hbm.at[pl.ds(base, n_idx)], idx_v)
    copies = []
    for j in range(n_full):
      copies.append(pltpu.async_copy(
          emb_hbm.at[idx_v.at[pl.ds(j * _GCHUNK, _GCHUNK)]],
          rows_v.at[pl.ds(j * _GCHUNK, _GCHUNK)], sem))
    if tail:
      copies.append(pltpu.async_copy(
          emb_hbm.at[idx_v.at[pl.ds(n_full * _GCHUNK, tail)]],
          rows_v.at[pl.ds(n_full * _GCHUNK, tail)], sem))
    for cp in copies:
      cp.wait()

    def row_body(r, carry):
      acc = rows_v[r * C]
      for c in range(1, C):
        acc = acc + rows_v[r * C + c]
      acc_v[r] = acc
      return carry

    lax.fori_loop(0, per_w, row_body, 0)
    pltpu.sync_copy(acc_v, out_hbm.at[pl.ds(wid * per_w, per_w)])

  return gather_sum(emb, idx_flat)


def _logits_tile(x, w, bvec):
  return lax.dot_general(
      x, w, (((1,), (1,)), ((), ())),
      preferred_element_type=jnp.float32) + bvec


def _stats0_body(x_ref, w_ref, b_ref, lse_ref, m_ref, s_ref):
  j = pl.program_id(0)
  nj = pl.num_programs(0)
  logits = _logits_tile(x_ref[...], w_ref[...], b_ref[...])
  tmax = jnp.max(logits, axis=1, keepdims=True)

  @pl.when(j == 0)
  def _():
    m_ref[...] = jnp.full_like(m_ref[...], -jnp.inf)
    s_ref[...] = jnp.zeros_like(s_ref[...])

  m_old = m_ref[...]
  m_new = jnp.maximum(m_old, tmax)
  s_ref[...] = (s_ref[...] * jnp.exp(m_old - m_new)
                + jnp.sum(jnp.exp(logits - m_new), axis=1, keepdims=True))
  m_ref[...] = m_new

  @pl.when(j == nj - 1)
  def _():
    lse_ref[...] = jnp.broadcast_to(
        m_ref[...] + jnp.log(s_ref[...]), lse_ref.shape)


def _make_fused_body(CB):
  def fused_body(x_ref, w_ref, b_ref, lse0_ref, o_ref, m2_ref, s2_ref):
    q = pl.program_id(0)
    j = pl.program_id(1)
    nchunk = pl.num_programs(0)
    w = w_ref[...]
    bvec = b_ref[...]

    @pl.when(jnp.logical_and(q == 0, j == 0))
    def _():
      m2_ref[0] = lse0_ref[:, 0:1]
      s2_ref[0] = jnp.ones((CB, 1), jnp.float32)

    @pl.when(q < nchunk - 1)
    def _stats():
      xs = x_ref[pl.ds((q + 1) * CB, CB), :]
      logits = _logits_tile(xs, w, bvec)
      tmax = jnp.max(logits, axis=1, keepdims=True)
      slot = lax.rem(q + 1, 2)

      @pl.when(j == 0)
      def _():
        m2_ref[slot] = jnp.full((CB, 1), -jnp.inf, jnp.float32)
        s2_ref[slot] = jnp.zeros((CB, 1), jnp.float32)

      m_old = m2_ref[slot]
      m_new = jnp.maximum(m_old, tmax)
      s2_ref[slot] = (s2_ref[slot] * jnp.exp(m_old - m_new)
                      + jnp.sum(jnp.exp(logits - m_new), axis=1,
                                keepdims=True))
      m2_ref[slot] = m_new

    slot_w = lax.rem(q, 2)
    xq = x_ref[pl.ds(q * CB, CB), :]
    logits_w = _logits_tile(xq, w, bvec)
    lse = m2_ref[slot_w] + jnp.log(s2_ref[slot_w])
    o_ref[...] = logits_w - lse

  return fused_body


def kernel(inputs, emb, W, b):
  B, C = inputs.shape
  V, D = emb.shape
  nvt = pl.cdiv(V, _VT)
  VP = nvt * _VT
  CB = B // _NCHUNK

  idx_flat = inputs.reshape(B * C).astype(jnp.int32)
  x = _gather_sum_sc(idx_flat, emb, B, C, D)

  W_pad = jnp.pad(W, ((0, VP - V), (0, 0)))
  b_pad = jnp.pad(b, (0, VP - V), constant_values=-1e30).reshape(1, VP)

  lse0 = pl.pallas_call(
      _stats0_body,
      grid=(nvt,),
      in_specs=[
          pl.BlockSpec((CB, D), lambda j: (0, 0)),
          pl.BlockSpec((_VT, D), lambda j: (j, 0)),
          pl.BlockSpec((1, _VT), lambda j: (0, j)),
      ],
      out_specs=pl.BlockSpec((CB, 128), lambda j: (0, 0)),
      out_shape=jax.ShapeDtypeStruct((CB, 128), jnp.float32),
      scratch_shapes=[
          pltpu.VMEM((CB, 1), jnp.float32),
          pltpu.VMEM((CB, 1), jnp.float32),
      ],
  )(x, W_pad, b_pad)

  log_probs = pl.pallas_call(
      _make_fused_body(CB),
      grid=(_NCHUNK, nvt),
      in_specs=[
          pl.BlockSpec((B, D), lambda q, j: (0, 0)),
          pl.BlockSpec((_VT, D), lambda q, j: (j, 0)),
          pl.BlockSpec((1, _VT), lambda q, j: (0, j)),
          pl.BlockSpec((CB, 128), lambda q, j: (0, 0)),
      ],
      out_specs=pl.BlockSpec((CB, _VT), lambda q, j: (q, j)),
      out_shape=jax.ShapeDtypeStruct((B, V), jnp.float32),
      scratch_shapes=[
          pltpu.VMEM((2, CB, 1), jnp.float32),
          pltpu.VMEM((2, CB, 1), jnp.float32),
      ],
  )(x, W_pad, b_pad, lse0)

  return log_probs


# speedup vs baseline: 1.6821x; 1.6821x over previous
import functools

import jax
import jax.numpy as jnp
from jax import lax
from jax.experimental import pallas as pl
from jax.experimental.pallas import tpu as pltpu
from jax.experimental.pallas import tpu_sc as plsc

_NUM_CORES = 2
_NUM_SUBCORES = 16
_NW = _NUM_CORES * _NUM_SUBCORES
_GCHUNK = 128

_VT = 2048
_NCHUNK = 4


def _gather_sum_sc(idx_flat, emb, B, C, D):
  per_w = B // _NW
  n_idx = per_w * C
  n_full = n_idx // _GCHUNK
  tail = n_idx - n_full * _GCHUNK

  mesh = plsc.VectorSubcoreMesh(
      core_axis_name="c", subcore_axis_name="s",
      num_cores=_NUM_CORES, num_subcores=_NUM_SUBCORES)

  @functools.partial(
      pl.kernel,
      out_type=jax.ShapeDtypeStruct((B, D), jnp.float32),
      mesh=mesh,
      compiler_params=pltpu.CompilerParams(use_tc_tiling_on_sc=False),
      scratch_types=[
          pltpu.VMEM((n_idx,), jnp.int32),
          pltpu.VMEM((n_idx, D), jnp.float32),
          pltpu.VMEM((per_w, D), jnp.float32),
          pltpu.SemaphoreType.DMA,
      ],
  )
  def gather_sum(emb_hbm, idx_hbm, out_hbm, idx_v, rows_v, acc_v, sem):
    wid = lax.axis_index("s") * _NUM_CORES + lax.axis_index("c")
    base = wid * n_idx
    pltpu.sync_copy(idx_hbm.at[pl.ds(base, n_idx)], idx_v)
    copies = []
    for j in range(n_full):
      copies.append(pltpu.async_copy(
          emb_hbm.at[idx_v.at[pl.ds(j * _GCHUNK, _GCHUNK)]],
          rows_v.at[pl.ds(j * _GCHUNK, _GCHUNK)], sem))
    if tail:
      copies.append(pltpu.async_copy(
          emb_hbm.at[idx_v.at[pl.ds(n_full * _GCHUNK, tail)]],
          rows_v.at[pl.ds(n_full * _GCHUNK, tail)], sem))
    for cp in copies:
      cp.wait()

    def row_body(r, carry):
      acc = rows_v[r * C]
      for c in range(1, C):
        acc = acc + rows_v[r * C + c]
      acc_v[r] = acc
      return carry

    lax.fori_loop(0, per_w, row_body, 0)
    pltpu.sync_copy(acc_v, out_hbm.at[pl.ds(wid * per_w, per_w)])

  return gather_sum(emb, idx_flat)


def _logits_tile(x, w, bvec):
  return lax.dot_general(
      x, w, (((1,), (1,)), ((), ())),
      preferred_element_type=jnp.float32) + bvec


def _make_stats0_body(CB):
  def stats0_body(x_ref, w_ref, b_ref, lse_ref, m_ref, s_ref):
    j = pl.program_id(0)
    nj = pl.num_programs(0)
    w = w_ref[pl.ds(j * _VT, _VT), :]
    bv = b_ref[pl.ds(j, 1), :]
    logits = _logits_tile(x_ref[...], w, bv)
    tmax = jnp.max(logits, axis=1, keepdims=True)

    @pl.when(j == 0)
    def _():
      m_ref[...] = jnp.full_like(m_ref[...], -jnp.inf)
      s_ref[...] = jnp.zeros_like(s_ref[...])

    m_old = m_ref[...]
    m_new = jnp.maximum(m_old, tmax)
    s_ref[...] = (s_ref[...] * jnp.exp(m_old - m_new)
                  + jnp.sum(jnp.exp(logits - m_new), axis=1, keepdims=True))
    m_ref[...] = m_new

    @pl.when(j == nj - 1)
    def _():
      lse_ref[...] = jnp.broadcast_to(
          m_ref[...] + jnp.log(s_ref[...]), lse_ref.shape)

  return stats0_body


def _make_fused_body(CB):
  def fused_body(x_ref, w_ref, b_ref, lse0_ref, o_ref,
                 m_ref, s_ref, lse_ref):
    q = pl.program_id(0)
    j = pl.program_id(1)
    nchunk = pl.num_programs(0)
    w = w_ref[pl.ds(j * _VT, _VT), :]
    bv = b_ref[pl.ds(j, 1), :]

    @pl.when(j == 0)
    def _():
      prev = m_ref[...] + jnp.log(s_ref[...])
      lse_ref[...] = jnp.where(q == 0, lse0_ref[:, 0:1], prev)
      m_ref[...] = jnp.full_like(m_ref[...], -jnp.inf)
      s_ref[...] = jnp.zeros_like(s_ref[...])

    @pl.when(q < nchunk - 1)
    def _stats():
      row = jnp.minimum(q + 1, nchunk - 1) * CB
      xs = x_ref[pl.ds(row, CB), :]
      logits = _logits_tile(xs, w, bv)
      tmax = jnp.max(logits, axis=1, keepdims=True)
      m_old = m_ref[...]
      m_new = jnp.maximum(m_old, tmax)
      s_ref[...] = (s_ref[...] * jnp.exp(m_old - m_new)
                    + jnp.sum(jnp.exp(logits - m_new), axis=1,
                              keepdims=True))
      m_ref[...] = m_new

    xq = x_ref[pl.ds(q * CB, CB), :]
    logits_w = _logits_tile(xq, w, bv)
    o_ref[...] = logits_w - lse_ref[...]

  return fused_body


def kernel(inputs, emb, W, b):
  B, C = inputs.shape
  V, D = emb.shape
  nvt = pl.cdiv(V, _VT)
  VP = nvt * _VT
  CB = B // _NCHUNK

  idx_flat = inputs.reshape(B * C).astype(jnp.int32)
  x = _gather_sum_sc(idx_flat, emb, B, C, D)

  W_pad = jnp.pad(W, ((0, VP - V), (0, 0)))
  b_pad = jnp.pad(b, (0, VP - V), constant_values=-1e30).reshape(nvt, _VT)

  lse0 = pl.pallas_call(
      _make_stats0_body(CB),
      grid=(nvt,),
      in_specs=[
          pl.BlockSpec((CB, D), lambda j: (0, 0)),
          pl.BlockSpec((VP, D), lambda j: (0, 0)),
          pl.BlockSpec((nvt, _VT), lambda j: (0, 0)),
      ],
      out_specs=pl.BlockSpec((CB, 128), lambda j: (0, 0)),
      out_shape=jax.ShapeDtypeStruct((CB, 128), jnp.float32),
      scratch_shapes=[
          pltpu.VMEM((CB, 1), jnp.float32),
          pltpu.VMEM((CB, 1), jnp.float32),
      ],
  )(x, W_pad, b_pad)

  log_probs = pl.pallas_call(
      _make_fused_body(CB),
      grid=(_NCHUNK, nvt),
      in_specs=[
          pl.BlockSpec((B, D), lambda q, j: (0, 0)),
          pl.BlockSpec((VP, D), lambda q, j: (0, 0)),
          pl.BlockSpec((nvt, _VT), lambda q, j: (0, 0)),
          pl.BlockSpec((CB, 128), lambda q, j: (0, 0)),
      ],
      out_specs=pl.BlockSpec((CB, _VT), lambda q, j: (q, j)),
      out_shape=jax.ShapeDtypeStruct((B, V), jnp.float32),
      scratch_shapes=[
          pltpu.VMEM((CB, 1), jnp.float32),
          pltpu.VMEM((CB, 1), jnp.float32),
          pltpu.VMEM((CB, 1), jnp.float32),
      ],
  )(x, W_pad, b_pad, lse0)

  return log_probs
